# Initial kernel scaffold; baseline (speedup 1.0000x reference)
import jax
import jax.numpy as jnp
from jax.experimental import pallas as pl


def kernel(node_feats, edge_feats, global_feats, edge_index, batch, W):
    raise NotImplementedError("write your pallas kernel here")


# speedup vs baseline: 6.8975x; 6.8975x over previous
import functools

import jax
import jax.numpy as jnp
from jax import lax
from jax.experimental import pallas as pl
from jax.experimental.pallas import tpu as pltpu
from jax.experimental.pallas import tpu_sc as plsc

N_NODES = 10000
N_EDGES = 320000
D_NODE = 128
D_EDGE = 16
D_GLOBAL = 16
N_GRAPHS = 64
HIDDEN = 128

_NW = 32
_PER_W = N_EDGES // _NW
_CHUNK = 80
_NCHUNK = _PER_W // _CHUNK

_LOG2 = 0.6931471805599453


def _proj_body(node_ref, batchf_ref, g_ref, w1_ref, w2_ref, w4_ref,
               a_ref, b_ref):
    gproj = jnp.dot(g_ref[...], w4_ref[...],
                    preferred_element_type=jnp.float32)
    iota = lax.broadcasted_iota(jnp.int32, (N_NODES, N_GRAPHS), 1)
    onehot = (batchf_ref[...] == iota).astype(jnp.float32)
    a_ref[...] = (
        jnp.dot(node_ref[...], w1_ref[...], preferred_element_type=jnp.float32)
        + jnp.dot(onehot, gproj, preferred_element_type=jnp.float32))
    b_ref[...] = jnp.dot(node_ref[...], w2_ref[...],
                         preferred_element_type=jnp.float32)


def _build_tables(node_feats, batch_f, global_feats, w1t, w2t, w4t):
    return pl.pallas_call(
        _proj_body,
        out_shape=[
            jax.ShapeDtypeStruct((N_NODES, HIDDEN), jnp.float32),
            jax.ShapeDtypeStruct((N_NODES, HIDDEN), jnp.float32),
        ],
    )(node_feats, batch_f, global_feats, w1t, w2t, w4t)


def _sc_body(a_hbm, b_hbm, src_hbm, dst_hbm, out_hbm,
             idx_s, idx_d, buf_a, buf_b, sem_a, sem_b):
    wid = lax.axis_index("s") * 2 + lax.axis_index("c")
    wbase = wid * _PER_W

    def chunk(j, carry):
        base = wbase + j * _CHUNK
        pltpu.sync_copy(src_hbm.at[pl.ds(base, _CHUNK)], idx_s)
        pltpu.sync_copy(dst_hbm.at[pl.ds(base, _CHUNK)], idx_d)
        ca = pltpu.async_copy(a_hbm.at[idx_s], buf_a, sem_a)
        cb = pltpu.async_copy(b_hbm.at[idx_d], buf_b, sem_b)
        ca.wait()
        cb.wait()

        def row(r, carry2):
            for c in range(HIDDEN // 16):
                sl = pl.ds(c * 16, 16)
                plsc.addupdate(buf_a.at[r, sl], buf_b[r, sl])
            return carry2

        lax.fori_loop(0, _CHUNK, row, 0)
        pltpu.sync_copy(buf_a, out_hbm.at[pl.ds(base, _CHUNK)])
        return carry

    lax.fori_loop(0, _NCHUNK, chunk, 0)


def _gather_add(a_tbl, b_tbl, src, dst):
    mesh = plsc.VectorSubcoreMesh(core_axis_name="c", subcore_axis_name="s")
    fn = functools.partial(
        pl.kernel,
        out_type=jax.ShapeDtypeStruct((N_EDGES, HIDDEN), jnp.float32),
        mesh=mesh,
        scratch_types=[
            pltpu.VMEM((_CHUNK,), jnp.int32),
            pltpu.VMEM((_CHUNK,), jnp.int32),
            pltpu.VMEM((_CHUNK, HIDDEN), jnp.float32),
            pltpu.VMEM((_CHUNK, HIDDEN), jnp.float32),
            pltpu.SemaphoreType.DMA,
            pltpu.SemaphoreType.DMA,
        ],
    )(_sc_body)
    return fn(a_tbl, b_tbl, src, dst)


_BLK = 2000


def _final_body(s_ref, ef_ref, w3_ref, o_ref):
    e = jnp.dot(ef_ref[...], w3_ref[...], preferred_element_type=jnp.float32)
    h = s_ref[...] + e
    o_ref[...] = (jnp.maximum(h, 0.0)
                  + jnp.log1p(jnp.exp(-jnp.abs(h))) - _LOG2)


def _finalize(s, edge_feats, w3t):
    grid = (N_EDGES // _BLK,)
    return pl.pallas_call(
        _final_body,
        grid=grid,
        in_specs=[
            pl.BlockSpec((_BLK, HIDDEN), lambda i: (i, 0)),
            pl.BlockSpec((_BLK, D_EDGE), lambda i: (i, 0)),
            pl.BlockSpec((D_EDGE, HIDDEN), lambda i: (0, 0)),
        ],
        out_specs=pl.BlockSpec((_BLK, HIDDEN), lambda i: (i, 0)),
        out_shape=jax.ShapeDtypeStruct((N_EDGES, HIDDEN), jnp.float32),
    )(s, edge_feats, w3t)


def kernel(node_feats, edge_feats, global_feats, edge_index, batch, W):
    wt = W.T
    w1t = wt[:D_NODE]
    w2t = wt[D_NODE:2 * D_NODE]
    w3t = wt[2 * D_NODE:2 * D_NODE + D_EDGE]
    w4t = wt[2 * D_NODE + D_EDGE:]
    batch_f = batch.astype(jnp.int32)[:, None]
    src = edge_index[0].astype(jnp.int32)
    dst = edge_index[1].astype(jnp.int32)

    a_tbl, b_tbl = _build_tables(node_feats, batch_f, global_feats,
                                 w1t, w2t, w4t)
    s = _gather_add(a_tbl, b_tbl, src, dst)
    return _finalize(s, edge_feats, w3t)


# speedup vs baseline: 9.8232x; 1.4242x over previous
import functools

import jax
import jax.numpy as jnp
from jax import lax
from jax.experimental import pallas as pl
from jax.experimental.pallas import tpu as pltpu
from jax.experimental.pallas import tpu_sc as plsc

N_NODES = 10000
N_EDGES = 320000
D_NODE = 128
D_EDGE = 16
D_GLOBAL = 16
N_GRAPHS = 64
HIDDEN = 128

_NW = 32
_PER_W = N_EDGES // _NW
_CHUNK = 80
_NCHUNK = _PER_W // _CHUNK

_LOG2 = 0.6931471805599453


def _proj_body(node_ref, batchf_ref, g_ref, w1_ref, w2_ref, w4_ref,
               a_ref, b_ref):
    gproj = jnp.dot(g_ref[...], w4_ref[...],
                    preferred_element_type=jnp.float32)
    iota = lax.broadcasted_iota(jnp.int32, (N_NODES, N_GRAPHS), 1)
    onehot = (batchf_ref[...] == iota).astype(jnp.float32)
    a_ref[...] = (
        jnp.dot(node_ref[...], w1_ref[...], preferred_element_type=jnp.float32)
        + jnp.dot(onehot, gproj, preferred_element_type=jnp.float32))
    b_ref[...] = jnp.dot(node_ref[...], w2_ref[...],
                         preferred_element_type=jnp.float32)


def _build_tables(node_feats, batch_f, global_feats, w1t, w2t, w4t):
    return pl.pallas_call(
        _proj_body,
        out_shape=[
            jax.ShapeDtypeStruct((N_NODES, HIDDEN), jnp.float32),
            jax.ShapeDtypeStruct((N_NODES, HIDDEN), jnp.float32),
        ],
    )(node_feats, batch_f, global_feats, w1t, w2t, w4t)


_NSLOT = 5


_NOUTER = _NCHUNK // _NSLOT


def _sc_body(a_hbm, b_hbm, src_hbm, dst_hbm, out_hbm,
             idx_s, idx_d, buf_a, buf_b,
             sem_a, sem_b, sem_st, sem_is, sem_id):
    wid = lax.axis_index("s") * 2 + lax.axis_index("c")
    wbase = wid * _PER_W

    def fire_idx(g, par):
        pltpu.async_copy(src_hbm.at[wid, g], idx_s.at[par], sem_is)
        pltpu.async_copy(dst_hbm.at[wid, g], idx_d.at[par], sem_id)

    def wait_idx(g, par):
        pltpu.make_async_copy(src_hbm.at[wid, g], idx_s.at[par],
                              sem_is).wait()
        pltpu.make_async_copy(dst_hbm.at[wid, g], idx_d.at[par],
                              sem_id).wait()

    def fire_gathers(par, b, slot):
        pltpu.async_copy(a_hbm.at[idx_s.at[par, b]], buf_a.at[slot],
                         sem_a.at[slot])
        pltpu.async_copy(b_hbm.at[idx_d.at[par, b]], buf_b.at[slot],
                         sem_b.at[slot])

    def wait_gathers(par, b, slot):
        pltpu.make_async_copy(a_hbm.at[idx_s.at[par, b]], buf_a.at[slot],
                              sem_a.at[slot]).wait()
        pltpu.make_async_copy(b_hbm.at[idx_d.at[par, b]], buf_b.at[slot],
                              sem_b.at[slot]).wait()

    def drain_store(slot):
        pltpu.make_async_copy(buf_a.at[slot], out_hbm.at[pl.ds(0, _CHUNK)],
                              sem_st.at[slot]).wait()

    fire_idx(0, 0)
    wait_idx(0, 0)
    fire_gathers(0, 0, 0)

    def outer(g, carry):
        par = lax.rem(g, 2)
        npar = 1 - par
        for b in range(_NSLOT):
            j = g * _NSLOT + b
            nslot = (b + 1) % _NSLOT

            if b == 0:
                @pl.when(g < _NOUTER - 1)
                def _():
                    fire_idx(g + 1, npar)

            @pl.when(j + 1 < _NCHUNK)
            def _():
                @pl.when(j + 1 >= _NSLOT)
                def _():
                    drain_store(nslot)
                if b == _NSLOT - 1:
                    wait_idx(g + 1, npar)
                    fire_gathers(npar, 0, nslot)
                else:
                    fire_gathers(par, b + 1, nslot)

            wait_gathers(par, b, b)

            def row(r, carry2):
                for c in range(HIDDEN // 16):
                    sl = pl.ds(c * 16, 16)
                    plsc.addupdate(buf_a.at[b, r, sl], buf_b[b, r, sl])
                return carry2

            lax.fori_loop(0, _CHUNK, row, 0)
            pltpu.async_copy(buf_a.at[b],
                             out_hbm.at[pl.ds(wbase + j * _CHUNK, _CHUNK)],
                             sem_st.at[b])
        return carry

    lax.fori_loop(0, _NOUTER, outer, 0)
    for s in range(_NSLOT):
        drain_store(s)


def _gather_add(a_tbl, b_tbl, src, dst):
    mesh = plsc.VectorSubcoreMesh(core_axis_name="c", subcore_axis_name="s")
    fn = functools.partial(
        pl.kernel,
        out_type=jax.ShapeDtypeStruct((N_EDGES, HIDDEN), jnp.float32),
        mesh=mesh,
        scratch_types=[
            pltpu.VMEM((2, _NSLOT, _CHUNK), jnp.int32),
            pltpu.VMEM((2, _NSLOT, _CHUNK), jnp.int32),
            pltpu.VMEM((_NSLOT, _CHUNK, HIDDEN), jnp.float32),
            pltpu.VMEM((_NSLOT, _CHUNK, HIDDEN), jnp.float32),
            pltpu.SemaphoreType.DMA((_NSLOT,)),
            pltpu.SemaphoreType.DMA((_NSLOT,)),
            pltpu.SemaphoreType.DMA((_NSLOT,)),
            pltpu.SemaphoreType.DMA,
            pltpu.SemaphoreType.DMA,
        ],
    )(_sc_body)
    src4 = src.reshape(_NW, _NOUTER, _NSLOT, _CHUNK)
    dst4 = dst.reshape(_NW, _NOUTER, _NSLOT, _CHUNK)
    return fn(a_tbl, b_tbl, src4, dst4)


_BLK = 2000


def _final_body(s_ref, ef_ref, w3_ref, o_ref):
    e = jnp.dot(ef_ref[...], w3_ref[...], preferred_element_type=jnp.float32)
    h = s_ref[...] + e
    o_ref[...] = (jnp.maximum(h, 0.0)
                  + jnp.log1p(jnp.exp(-jnp.abs(h))) - _LOG2)


def _finalize(s, edge_feats, w3t):
    grid = (N_EDGES // _BLK,)
    return pl.pallas_call(
        _final_body,
        grid=grid,
        in_specs=[
            pl.BlockSpec((_BLK, HIDDEN), lambda i: (i, 0)),
            pl.BlockSpec((_BLK, D_EDGE), lambda i: (i, 0)),
            pl.BlockSpec((D_EDGE, HIDDEN), lambda i: (0, 0)),
        ],
        out_specs=pl.BlockSpec((_BLK, HIDDEN), lambda i: (i, 0)),
        out_shape=jax.ShapeDtypeStruct((N_EDGES, HIDDEN), jnp.float32),
    )(s, edge_feats, w3t)


def kernel(node_feats, edge_feats, global_feats, edge_index, batch, W):
    wt = W.T
    w1t = wt[:D_NODE]
    w2t = wt[D_NODE:2 * D_NODE]
    w3t = wt[2 * D_NODE:2 * D_NODE + D_EDGE]
    w4t = wt[2 * D_NODE + D_EDGE:]
    batch_f = batch.astype(jnp.int32)[:, None]
    src = edge_index[0].astype(jnp.int32)
    dst = edge_index[1].astype(jnp.int32)

    a_tbl, b_tbl = _build_tables(node_feats, batch_f, global_feats,
                                 w1t, w2t, w4t)
    s = _gather_add(a_tbl, b_tbl, src, dst)
    return _finalize(s, edge_feats, w3t)
